# baseline (speedup 1.0000x reference)
import functools

import jax
import jax.numpy as jnp
from jax import lax
from jax.experimental import pallas as pl
from jax.experimental.pallas import tpu as pltpu
from jax.experimental.pallas import tpu_sc as plsc

_VOCAB = 100000
_EMB = 32
_WIN = 20
_BATCH = 1024

_NC = 2
_NS = 16
_NW = _NC * _NS
_BPW = _BATCH // _NW
_IPW = _BPW * _WIN
_CHUNK = 128
_NCHUNK = _IPW // _CHUNK


def _make_pooling_kernel():
    mesh = plsc.VectorSubcoreMesh(
        core_axis_name="c", subcore_axis_name="s",
        num_cores=_NC, num_subcores=_NS,
    )

    @functools.partial(
        pl.kernel,
        mesh=mesh,
        compiler_params=pltpu.CompilerParams(use_tc_tiling_on_sc=False),
        out_type=jax.ShapeDtypeStruct((_BATCH, _EMB), jnp.float32),
        scratch_types=[
            pltpu.VMEM((_NCHUNK, _CHUNK), jnp.int32),
            pltpu.VMEM((_IPW, _EMB), jnp.float32),
            pltpu.VMEM((_BPW, _EMB), jnp.float32),
            pltpu.SemaphoreType.DMA,
        ],
    )
    def pooling(ctx_hbm, table_hbm, pooled_hbm, idx_v, rows_v, pool_v, sem):
        wid = lax.axis_index("s") * _NC + lax.axis_index("c")
        pltpu.sync_copy(ctx_hbm.at[wid], idx_v)
        copies = [
            pltpu.async_copy(
                table_hbm.at[idx_v.at[c]],
                rows_v.at[pl.ds(c * _CHUNK, _CHUNK)],
                sem,
            )
            for c in range(_NCHUNK)
        ]
        for cp in copies:
            cp.wait()

        inv = jnp.float32(1.0 / _WIN)

        def pool_one(b, carry):
            p0 = b * _WIN
            a0 = rows_v[p0, 0:16]
            a1 = rows_v[p0, 16:32]
            for w in range(1, _WIN):
                a0 = a0 + rows_v[p0 + w, 0:16]
                a1 = a1 + rows_v[p0 + w, 16:32]
            pool_v[b, 0:16] = a0 * inv
            pool_v[b, 16:32] = a1 * inv
            return carry

        lax.fori_loop(0, _BPW, pool_one, 0)
        pltpu.sync_copy(pool_v, pooled_hbm.at[pl.ds(wid * _BPW, _BPW)])

    return pooling


@functools.lru_cache(maxsize=1)
def _get_pooling():
    return _make_pooling_kernel()

_TV = 512


def _proj_body(pooled_ref, w_ref, b_ref, out_ref):
    acc = lax.dot_general(
        pooled_ref[...], w_ref[...],
        dimension_numbers=(((1,), (1,)), ((), ())),
        preferred_element_type=jnp.float32,
    )
    out_ref[...] = acc + b_ref[...]


def _project(pooled, W, b2d):
    return pl.pallas_call(
        _proj_body,
        grid=(pl.cdiv(_VOCAB, _TV),),
        in_specs=[
            pl.BlockSpec((_BATCH, _EMB), lambda j: (0, 0)),
            pl.BlockSpec((_TV, _EMB), lambda j: (j, 0)),
            pl.BlockSpec((1, _TV), lambda j: (0, j)),
        ],
        out_specs=pl.BlockSpec((_BATCH, _TV), lambda j: (0, j)),
        out_shape=jax.ShapeDtypeStruct((_BATCH, _VOCAB), jnp.float32),
    )(pooled, W, b2d)


def kernel(context, emb_table, W, b):
    ctx = context.astype(jnp.int32).reshape(_NW, _NCHUNK, _CHUNK)
    pooled = _get_pooling()(ctx, emb_table)
    return _project(pooled, W, b.reshape(1, _VOCAB))


# speedup vs baseline: 2.6086x; 2.6086x over previous
import functools

import jax
import jax.numpy as jnp
from jax import lax
from jax.experimental import pallas as pl
from jax.experimental.pallas import tpu as pltpu
from jax.experimental.pallas import tpu_sc as plsc

_VOCAB = 100000
_EMB = 32
_WIN = 20
_BATCH = 1024

_NC = 2
_NS = 16
_NW = _NC * _NS
_BPW = _BATCH // _NW
_IPW = _BPW * _WIN
_CHUNK = 128
_NCHUNK = _IPW // _CHUNK


def _make_pooling_kernel():
    mesh = plsc.VectorSubcoreMesh(
        core_axis_name="c", subcore_axis_name="s",
        num_cores=_NC, num_subcores=_NS,
    )

    @functools.partial(
        pl.kernel,
        mesh=mesh,
        compiler_params=pltpu.CompilerParams(
            use_tc_tiling_on_sc=False, needs_layout_passes=False),
        out_type=jax.ShapeDtypeStruct((_EMB, _BATCH), jnp.float32),
        scratch_types=[
            pltpu.VMEM((_NCHUNK, _CHUNK), jnp.int32),
            pltpu.VMEM((_IPW, _EMB), jnp.float32),
            pltpu.VMEM((_EMB, _BPW), jnp.float32),
            pltpu.SemaphoreType.DMA,
        ],
    )
    def pooling(ctx_hbm, table_hbm, pooled_hbm, idx_v, rows_v, pool_v, sem):
        wid = lax.axis_index("s") * _NC + lax.axis_index("c")
        pltpu.sync_copy(ctx_hbm.at[wid], idx_v)
        copies = [
            pltpu.async_copy(
                table_hbm.at[idx_v.at[c]],
                rows_v.at[pl.ds(c * _CHUNK, _CHUNK)],
                sem,
            )
            for c in range(_NCHUNK)
        ]
        for cp in copies:
            cp.wait()

        inv = jnp.float32(1.0 / _WIN)
        d_lo = lax.iota(jnp.int32, 16)
        d_hi = d_lo + 16

        def pool_one(b, carry):
            p0 = b * _WIN
            a0 = rows_v[p0, 0:16]
            a1 = rows_v[p0, 16:32]
            for w in range(1, _WIN):
                a0 = a0 + rows_v[p0 + w, 0:16]
                a1 = a1 + rows_v[p0 + w, 16:32]
            bcol = jnp.full((16,), b, jnp.int32)
            plsc.store_scatter(pool_v, [d_lo, bcol], a0 * inv)
            plsc.store_scatter(pool_v, [d_hi, bcol], a1 * inv)
            return carry

        lax.fori_loop(0, _BPW, pool_one, 0)
        pltpu.sync_copy(pool_v, pooled_hbm.at[:, pl.ds(wid * _BPW, _BPW)])

    return pooling


@functools.lru_cache(maxsize=1)
def _get_pooling():
    return _make_pooling_kernel()


_TV = 2048


def _proj_body(wt_ref, pooled_ref, b_ref, out_ref):
    acc = lax.dot_general(
        wt_ref[...], pooled_ref[...],
        dimension_numbers=(((0,), (0,)), ((), ())),
        preferred_element_type=jnp.float32,
    )
    out_ref[...] = acc + b_ref[...]


def _project_t(wt, pooled_t, b2d):
    return pl.pallas_call(
        _proj_body,
        grid=(pl.cdiv(_VOCAB, _TV),),
        in_specs=[
            pl.BlockSpec((_EMB, _TV), lambda j: (0, j)),
            pl.BlockSpec((_EMB, _BATCH), lambda j: (0, 0)),
            pl.BlockSpec((_TV, 1), lambda j: (j, 0)),
        ],
        out_specs=pl.BlockSpec((_TV, _BATCH), lambda j: (j, 0)),
        out_shape=jax.ShapeDtypeStruct((_VOCAB, _BATCH), jnp.float32),
    )(wt, pooled_t, b2d)


def kernel(context, emb_table, W, b):
    ctx = context.astype(jnp.int32).reshape(_NW, _NCHUNK, _CHUNK)
    pooled_t = _get_pooling()(ctx, emb_table)
    out_t = _project_t(W.T, pooled_t, b.reshape(_VOCAB, 1))
    return out_t.T


# speedup vs baseline: 3.2127x; 1.2316x over previous
import functools

import jax
import jax.numpy as jnp
from jax import lax
from jax.experimental import pallas as pl
from jax.experimental.pallas import tpu as pltpu
from jax.experimental.pallas import tpu_sc as plsc

_VOCAB = 100000
_EMB = 32
_WIN = 20
_BATCH = 1024
_AUG = _EMB + 1

_NC = 2
_NS = 16
_NW = _NC * _NS
_BPW = _BATCH // _NW


def _make_pooling_kernel():
    mesh = plsc.VectorSubcoreMesh(
        core_axis_name="c", subcore_axis_name="s",
        num_cores=_NC, num_subcores=_NS,
    )

    @functools.partial(
        pl.kernel,
        mesh=mesh,
        compiler_params=pltpu.CompilerParams(
            use_tc_tiling_on_sc=False, needs_layout_passes=False),
        out_type=jax.ShapeDtypeStruct((_AUG, _BATCH), jnp.float32),
        scratch_types=[
            pltpu.VMEM((_WIN, _BPW), jnp.int32),
            pltpu.VMEM((_WIN, _BPW, _EMB), jnp.float32),
            pltpu.VMEM((_AUG, _BPW), jnp.float32),
            pltpu.SemaphoreType.DMA,
        ],
    )
    def pooling(ctx_hbm, table_hbm, pooled_hbm, idx_v, rows_v, pool_v, sem):
        wid = lax.axis_index("s") * _NC + lax.axis_index("c")
        pltpu.sync_copy(ctx_hbm.at[:, pl.ds(wid * _BPW, _BPW)], idx_v)
        copies = [
            pltpu.async_copy(
                table_hbm.at[idx_v.at[w]],
                rows_v.at[w],
                sem,
            )
            for w in range(_WIN)
        ]
        for cp in copies:
            cp.wait()

        inv = jnp.float32(1.0 / _WIN)
        one = jnp.full((16,), 1.0, jnp.float32)
        d_lo = lax.iota(jnp.int32, 16)
        d_hi = d_lo + 16

        def pool_one(b, carry):
            a0 = rows_v[0, b, 0:16]
            a1 = rows_v[0, b, 16:32]
            for w in range(1, _WIN):
                a0 = a0 + rows_v[w, b, 0:16]
                a1 = a1 + rows_v[w, b, 16:32]
            bcol = jnp.full((16,), b, jnp.int32)
            plsc.store_scatter(pool_v, [d_lo, bcol], a0 * inv)
            plsc.store_scatter(pool_v, [d_hi, bcol], a1 * inv)
            return carry

        lax.fori_loop(0, _BPW, pool_one, 0)
        pool_v[_EMB, 0:16] = one
        pool_v[_EMB, 16:32] = one
        pltpu.sync_copy(pool_v, pooled_hbm.at[:, pl.ds(wid * _BPW, _BPW)])

    return pooling


@functools.lru_cache(maxsize=1)
def _get_pooling():
    return _make_pooling_kernel()


_TV = 2048


def _proj_body(wt_ref, pooled_ref, out_ref):
    out_ref[...] = lax.dot_general(
        wt_ref[...], pooled_ref[...],
        dimension_numbers=(((0,), (0,)), ((), ())),
        preferred_element_type=jnp.float32,
    )


def _project_t(wt_aug, pooled_t):
    return pl.pallas_call(
        _proj_body,
        grid=(pl.cdiv(_VOCAB, _TV),),
        in_specs=[
            pl.BlockSpec((_AUG, _TV), lambda j: (0, j)),
            pl.BlockSpec((_AUG, _BATCH), lambda j: (0, 0)),
        ],
        out_specs=pl.BlockSpec((_TV, _BATCH), lambda j: (j, 0)),
        out_shape=jax.ShapeDtypeStruct((_VOCAB, _BATCH), jnp.float32),
    )(wt_aug, pooled_t)


def kernel(context, emb_table, W, b):
    ctx_t = context.T.astype(jnp.int32)
    pooled_t = _get_pooling()(ctx_t, emb_table)
    wt_aug = jnp.concatenate([W.T, b[None, :]], axis=0)
    out_t = _project_t(wt_aug, pooled_t)
    return out_t.T


# speedup vs baseline: 3.6507x; 1.1363x over previous
import functools

import jax
import jax.numpy as jnp
from jax import lax
from jax.experimental import pallas as pl
from jax.experimental.pallas import tpu as pltpu
from jax.experimental.pallas import tpu_sc as plsc

_VOCAB = 100000
_EMB = 32
_WIN = 20
_BATCH = 1024
_AUG = _EMB + 1

_NC = 2
_NS = 16
_NW = _NC * _NS


def _make_pooling_kernel():
    mesh = plsc.VectorSubcoreMesh(
        core_axis_name="c", subcore_axis_name="s",
        num_cores=_NC, num_subcores=_NS,
    )

    @functools.partial(
        pl.kernel,
        mesh=mesh,
        compiler_params=pltpu.CompilerParams(
            use_tc_tiling_on_sc=False, needs_layout_passes=False),
        out_type=jax.ShapeDtypeStruct((_AUG, _BATCH), jnp.float32),
        scratch_types=[
            pltpu.VMEM((_VOCAB,), jnp.float32),
            pltpu.VMEM((_WIN, _BATCH), jnp.int32),
            pltpu.VMEM((_BATCH,), jnp.float32),
            pltpu.SemaphoreType.DMA,
            pltpu.SemaphoreType.DMA,
        ],
    )
    def pooling(ctx_hbm, tablet_hbm, pooled_hbm, trow_v, idx_v, acc_v, s1, s2):
        wid = lax.axis_index("s") * _NC + lax.axis_index("c")
        cp1 = pltpu.async_copy(tablet_hbm.at[wid], trow_v, s1)
        cp2 = pltpu.async_copy(ctx_hbm, idx_v, s2)
        cp1.wait()
        cp2.wait()

        inv = jnp.float32(1.0 / _WIN)

        def chunk(ic, carry):
            base = ic * 16
            acc = plsc.load_gather(trow_v, [idx_v[0, pl.ds(base, 16)]])
            for w in range(1, _WIN):
                acc = acc + plsc.load_gather(trow_v, [idx_v[w, pl.ds(base, 16)]])
            acc_v[pl.ds(base, 16)] = acc * inv
            return carry

        lax.fori_loop(0, _BATCH // 16, chunk, 0)
        pltpu.sync_copy(acc_v, pooled_hbm.at[wid])

        @pl.when(wid == 0)
        def _ones_row():
            one = jnp.full((16,), 1.0, jnp.float32)

            def ones_chunk(ic, carry):
                acc_v[pl.ds(ic * 16, 16)] = one
                return carry

            lax.fori_loop(0, _BATCH // 16, ones_chunk, 0)
            pltpu.sync_copy(acc_v, pooled_hbm.at[_EMB])

    return pooling


@functools.lru_cache(maxsize=1)
def _get_pooling():
    return _make_pooling_kernel()


_TV = 2048


def _proj_body(wt_ref, pooled_ref, out_ref):
    out_ref[...] = lax.dot_general(
        wt_ref[...], pooled_ref[...],
        dimension_numbers=(((0,), (0,)), ((), ())),
        preferred_element_type=jnp.float32,
    )


def _project_t(wt_aug, pooled_t):
    return pl.pallas_call(
        _proj_body,
        grid=(pl.cdiv(_VOCAB, _TV),),
        in_specs=[
            pl.BlockSpec((_AUG, _TV), lambda j: (0, j)),
            pl.BlockSpec((_AUG, _BATCH), lambda j: (0, 0)),
        ],
        out_specs=pl.BlockSpec((_TV, _BATCH), lambda j: (j, 0)),
        out_shape=jax.ShapeDtypeStruct((_VOCAB, _BATCH), jnp.float32),
    )(wt_aug, pooled_t)


def kernel(context, emb_table, W, b):
    ctx_t = context.T.astype(jnp.int32)
    table_t = emb_table.T
    pooled_t = _get_pooling()(ctx_t, table_t)
    wt_aug = jnp.concatenate([W.T, b[None, :]], axis=0)
    out_t = _project_t(wt_aug, pooled_t)
    return out_t.T


# speedup vs baseline: 3.8062x; 1.0426x over previous
import functools

import jax
import jax.numpy as jnp
from jax import lax
from jax.experimental import pallas as pl
from jax.experimental.pallas import tpu as pltpu
from jax.experimental.pallas import tpu_sc as plsc

_VOCAB = 100000
_EMB = 32
_WIN = 20
_BATCH = 1024
_AUG = _EMB + 1

_NC = 2
_NS = 16
_NW = _NC * _NS


def _make_pooling_kernel():
    mesh = plsc.VectorSubcoreMesh(
        core_axis_name="c", subcore_axis_name="s",
        num_cores=_NC, num_subcores=_NS,
    )

    @functools.partial(
        pl.kernel,
        mesh=mesh,
        compiler_params=pltpu.CompilerParams(
            use_tc_tiling_on_sc=False, needs_layout_passes=False),
        out_type=jax.ShapeDtypeStruct((_EMB, _BATCH), jnp.float32),
        scratch_types=[
            pltpu.VMEM((_VOCAB,), jnp.float32),
            pltpu.VMEM((_WIN, _BATCH), jnp.int32),
            pltpu.VMEM((_BATCH,), jnp.float32),
            pltpu.SemaphoreType.DMA,
            pltpu.SemaphoreType.DMA,
        ],
    )
    def pooling(ctx_hbm, tablet_hbm, pooled_hbm, trow_v, idx_v, acc_v, s1, s2):
        wid = lax.axis_index("s") * _NC + lax.axis_index("c")
        cp1 = pltpu.async_copy(tablet_hbm.at[wid], trow_v, s1)
        cp2 = pltpu.async_copy(ctx_hbm, idx_v, s2)
        cp1.wait()
        cp2.wait()

        inv = jnp.float32(1.0 / _WIN)

        def chunk(ic, carry):
            base = ic * 16
            acc = plsc.load_gather(trow_v, [idx_v[0, pl.ds(base, 16)]])
            for w in range(1, _WIN):
                acc = acc + plsc.load_gather(trow_v, [idx_v[w, pl.ds(base, 16)]])
            acc_v[pl.ds(base, 16)] = acc * inv
            return carry

        lax.fori_loop(0, _BATCH // 16, chunk, 0)
        pltpu.sync_copy(acc_v, pooled_hbm.at[wid])

    return pooling


@functools.lru_cache(maxsize=1)
def _get_pooling():
    return _make_pooling_kernel()


_TV = 2048


def _proj_body(wt_ref, pooled_ref, b_ref, out_ref):
    acc = lax.dot_general(
        wt_ref[...], pooled_ref[...],
        dimension_numbers=(((0,), (0,)), ((), ())),
        preferred_element_type=jnp.float32,
    )
    ones_row = jnp.ones((1, _BATCH), jnp.float32)
    bias = lax.dot_general(
        b_ref[...], ones_row,
        dimension_numbers=(((0,), (0,)), ((), ())),
        preferred_element_type=jnp.float32,
    )
    out_ref[...] = acc + bias


def _project_t(wt, pooled_t, b2d):
    return pl.pallas_call(
        _proj_body,
        grid=(pl.cdiv(_VOCAB, _TV),),
        in_specs=[
            pl.BlockSpec((_EMB, _TV), lambda j: (0, j)),
            pl.BlockSpec((_EMB, _BATCH), lambda j: (0, 0)),
            pl.BlockSpec((1, _TV), lambda j: (0, j)),
        ],
        out_specs=pl.BlockSpec((_TV, _BATCH), lambda j: (j, 0)),
        out_shape=jax.ShapeDtypeStruct((_VOCAB, _BATCH), jnp.float32),
    )(wt, pooled_t, b2d)


def kernel(context, emb_table, W, b):
    ctx_t = context.T.astype(jnp.int32)
    table_t = emb_table.T
    pooled_t = _get_pooling()(ctx_t, table_t)
    out_t = _project_t(W.T, pooled_t, b.reshape(1, _VOCAB))
    return out_t.T


# speedup vs baseline: 3.8066x; 1.0001x over previous
import functools

import jax
import jax.numpy as jnp
from jax import lax
from jax.experimental import pallas as pl
from jax.experimental.pallas import tpu as pltpu
from jax.experimental.pallas import tpu_sc as plsc

_VOCAB = 100000
_EMB = 32
_WIN = 20
_BATCH = 1024
_AUG = _EMB + 1

_NC = 2
_NS = 16
_NW = _NC * _NS


def _make_pooling_kernel():
    mesh = plsc.VectorSubcoreMesh(
        core_axis_name="c", subcore_axis_name="s",
        num_cores=_NC, num_subcores=_NS,
    )

    @functools.partial(
        pl.kernel,
        mesh=mesh,
        compiler_params=pltpu.CompilerParams(
            use_tc_tiling_on_sc=False, needs_layout_passes=False),
        out_type=jax.ShapeDtypeStruct((_EMB, _BATCH), jnp.float32),
        scratch_types=[
            pltpu.VMEM((_VOCAB,), jnp.float32),
            pltpu.VMEM((_WIN, _BATCH), jnp.int32),
            pltpu.VMEM((_BATCH,), jnp.float32),
            pltpu.SemaphoreType.DMA,
            pltpu.SemaphoreType.DMA,
        ],
    )
    def pooling(ctx_hbm, tablet_hbm, pooled_hbm, trow_v, idx_v, acc_v, s1, s2):
        wid = lax.axis_index("s") * _NC + lax.axis_index("c")
        cp1 = pltpu.async_copy(tablet_hbm.at[wid], trow_v, s1)
        cp2 = pltpu.async_copy(ctx_hbm, idx_v, s2)
        cp1.wait()
        cp2.wait()

        inv = jnp.float32(1.0 / _WIN)

        def chunk(ic, carry):
            base = ic * 16
            acc = plsc.load_gather(trow_v, [idx_v[0, pl.ds(base, 16)]])
            for w in range(1, _WIN):
                acc = acc + plsc.load_gather(trow_v, [idx_v[w, pl.ds(base, 16)]])
            acc_v[pl.ds(base, 16)] = acc * inv
            return carry

        lax.fori_loop(0, _BATCH // 16, chunk, 0)
        pltpu.sync_copy(acc_v, pooled_hbm.at[wid])

    return pooling


@functools.lru_cache(maxsize=1)
def _get_pooling():
    return _make_pooling_kernel()


_TV = 4096


def _proj_body(wt_ref, pooled_ref, b_ref, out_ref):
    acc = lax.dot_general(
        wt_ref[...], pooled_ref[...],
        dimension_numbers=(((0,), (0,)), ((), ())),
        preferred_element_type=jnp.float32,
    )
    ones_row = jnp.ones((1, _BATCH), jnp.float32)
    bias = lax.dot_general(
        b_ref[...], ones_row,
        dimension_numbers=(((0,), (0,)), ((), ())),
        preferred_element_type=jnp.float32,
    )
    out_ref[...] = acc + bias


def _project_t(wt, pooled_t, b2d):
    return pl.pallas_call(
        _proj_body,
        grid=(pl.cdiv(_VOCAB, _TV),),
        in_specs=[
            pl.BlockSpec((_EMB, _TV), lambda j: (0, j)),
            pl.BlockSpec((_EMB, _BATCH), lambda j: (0, 0)),
            pl.BlockSpec((1, _TV), lambda j: (0, j)),
        ],
        out_specs=pl.BlockSpec((_TV, _BATCH), lambda j: (j, 0)),
        out_shape=jax.ShapeDtypeStruct((_VOCAB, _BATCH), jnp.float32),
    )(wt, pooled_t, b2d)


def kernel(context, emb_table, W, b):
    ctx_t = context.T.astype(jnp.int32)
    table_t = emb_table.T
    pooled_t = _get_pooling()(ctx_t, table_t)
    out_t = _project_t(W.T, pooled_t, b.reshape(1, _VOCAB))
    return out_t.T


# speedup vs baseline: 4.2384x; 1.1134x over previous
import functools

import jax
import jax.numpy as jnp
from jax import lax
from jax.experimental import pallas as pl
from jax.experimental.pallas import tpu as pltpu
from jax.experimental.pallas import tpu_sc as plsc

_VOCAB = 100000
_EMB = 32
_WIN = 20
_BATCH = 1024
_AUG = _EMB + 1

_NC = 2
_NS = 16
_NW = _NC * _NS


def _make_pooling_kernel():
    mesh = plsc.VectorSubcoreMesh(
        core_axis_name="c", subcore_axis_name="s",
        num_cores=_NC, num_subcores=_NS,
    )

    @functools.partial(
        pl.kernel,
        mesh=mesh,
        compiler_params=pltpu.CompilerParams(needs_layout_passes=False),
        out_type=jax.ShapeDtypeStruct((_EMB, _BATCH), jnp.float32),
        scratch_types=[
            pltpu.VMEM((_VOCAB,), jnp.float32),
            pltpu.VMEM((_WIN, _BATCH), jnp.int32),
            pltpu.VMEM((_BATCH,), jnp.float32),
            pltpu.SemaphoreType.DMA,
            pltpu.SemaphoreType.DMA,
        ],
    )
    def pooling(ctx_hbm, tablet_hbm, pooled_hbm, trow_v, idx_v, acc_v, s1, s2):
        wid = lax.axis_index("s") * _NC + lax.axis_index("c")
        cp1 = pltpu.async_copy(tablet_hbm.at[wid], trow_v, s1)
        cp2 = pltpu.async_copy(ctx_hbm, idx_v, s2)
        cp1.wait()
        cp2.wait()

        inv = jnp.float32(1.0 / _WIN)

        def chunk(ic, carry):
            base = ic * 16
            acc = plsc.load_gather(trow_v, [idx_v[0, pl.ds(base, 16)]])
            for w in range(1, _WIN):
                acc = acc + plsc.load_gather(trow_v, [idx_v[w, pl.ds(base, 16)]])
            acc_v[pl.ds(base, 16)] = acc * inv
            return carry

        lax.fori_loop(0, _BATCH // 16, chunk, 0)
        pltpu.sync_copy(acc_v, pooled_hbm.at[wid])

    return pooling


@functools.lru_cache(maxsize=1)
def _get_pooling():
    return _make_pooling_kernel()


_TV = 2048


def _proj_body(wt_ref, pooled_ref, b_ref, out_ref):
    acc = lax.dot_general(
        wt_ref[...], pooled_ref[...],
        dimension_numbers=(((0,), (0,)), ((), ())),
        preferred_element_type=jnp.float32,
    )
    ones_row = jnp.ones((1, _BATCH), jnp.float32)
    bias = lax.dot_general(
        b_ref[...], ones_row,
        dimension_numbers=(((0,), (0,)), ((), ())),
        preferred_element_type=jnp.float32,
    )
    out_ref[...] = acc + bias


def _project_t(wt, pooled_t, b2d):
    return pl.pallas_call(
        _proj_body,
        grid=(pl.cdiv(_VOCAB, _TV),),
        in_specs=[
            pl.BlockSpec((_EMB, _TV), lambda j: (0, j)),
            pl.BlockSpec((_EMB, _BATCH), lambda j: (0, 0)),
            pl.BlockSpec((1, _TV), lambda j: (0, j)),
        ],
        out_specs=pl.BlockSpec((_TV, _BATCH), lambda j: (j, 0)),
        out_shape=jax.ShapeDtypeStruct((_VOCAB, _BATCH), jnp.float32),
    )(wt, pooled_t, b2d)


def kernel(context, emb_table, W, b):
    ctx_t = context.T.astype(jnp.int32)
    table_t = emb_table.T
    pooled_t = _get_pooling()(ctx_t, table_t)
    out_t = _project_t(W.T, pooled_t, b.reshape(1, _VOCAB))
    return out_t.T
